# baseline (speedup 1.0000x reference)
import functools

import jax
import jax.numpy as jnp
from jax import lax
from jax.experimental import pallas as pl
from jax.experimental.pallas import tpu as pltpu
from jax.experimental.pallas import tpu_sc as plsc

B = 16384
D = 64
V = 1000000
NUM_CORES = 2
NUM_SUBCORES = 16
NW = NUM_CORES * NUM_SUBCORES
NV = D // 16
NSTRIP = (V + 127) // 128
SPW = (NSTRIP + NW - 1) // NW
LISTCAP = B + 32
SORTCAP = B + 16 * (SPW + 1)
NBUF = 4
NBLK = B // 16
FILLCAP = 256


def _tower_body(ids_hbm, table_hbm, out_hbm,
                allids_v, lid_v, lb_v, sid_v, sb_v, strips_v, rowbuf_v,
                fill_v, cnt_s, off_s,
                sem_strip, sem_out):
    c = lax.axis_index("c")
    s = lax.axis_index("s")
    wid = s * NUM_CORES + c
    s0 = wid * SPW

    pltpu.sync_copy(ids_hbm, allids_v)

    lanes = lax.iota(jnp.int32, 16)
    perms = [lanes ^ (1 << k) for k in range(4)]

    zeros16 = jnp.broadcast_to(jnp.int32(0), (16,))
    ones16 = jnp.broadcast_to(jnp.int32(1), (16,))
    for i in range(FILLCAP // 16):
        fill_v[pl.ds(16 * i, 16)] = zeros16

    def scan_blk(g, k):
        idvec = allids_v[pl.ds(g * 16, 16)]
        stripv = idvec >> 7
        m = (stripv >= s0) & (stripv < s0 + SPW)
        stl = jnp.where(m, stripv - s0, jnp.int32(SPW))
        plsc.addupdate_scatter(fill_v, [stl], ones16, mask=m)
        skeys, svals, om = plsc.sort_key_val(idvec, lanes + g * 16, mask=m)
        cnt = plsc.all_reduce_population_count(m)
        c = cnt if getattr(cnt, "ndim", 0) == 0 else cnt[0]
        plsc.store_scatter(lid_v, [k + lanes], skeys, mask=om)
        plsc.store_scatter(lb_v, [k + lanes], svals, mask=om)
        return k + c
    num_pairs = lax.fori_loop(0, NBLK, scan_blk, jnp.int32(0), unroll=8)

    nblk_pairs = (num_pairs + 15) >> 4

    def offs_blk(kb, running):
        cvec = fill_v[pl.ds(kb * 16, 16)]
        for l in range(16):
            st = kb * 16 + l
            @pl.when(st < SPW)
            def _():
                cnt_s[st] = cvec[l]
                off_s[st] = running
            plsc.store_scatter(
                fill_v, [jnp.broadcast_to(st, (16,)).astype(jnp.int32)],
                jnp.broadcast_to(running, (16,)), mask=lanes == 0)
            running = jnp.where(
                st < SPW,
                running + ((cvec[l] + 15) & ~jnp.int32(15)),
                running,
            )
        return running
    lax.fori_loop(0, (SPW + 15) // 16, offs_blk, jnp.int32(0))

    def sort_blk(kb, carry):
        idblk = lid_v[pl.ds(kb * 16, 16)]
        bblk = lb_v[pl.ds(kb * 16, 16)]
        valid = (kb * 16 + lanes) < num_pairs
        stl = jnp.where(valid, (idblk >> 7) - s0, jnp.int32(SPW))
        fill = plsc.load_gather(fill_v, [stl])
        rank = zeros16
        for k2 in range(1, 16):
            prev = stl.at[jnp.maximum(lanes - k2, 0)].get(
                mode="promise_in_bounds")
            eq = (prev == stl) & (lanes >= k2)
            rank = rank + jnp.where(eq, jnp.int32(1), jnp.int32(0))
        pos = fill + rank
        plsc.store_scatter(sid_v, [pos], idblk, mask=valid)
        plsc.store_scatter(sb_v, [pos], bblk, mask=valid)
        plsc.addupdate_scatter(fill_v, [stl], ones16, mask=valid)
        return carry
    lax.fori_loop(0, nblk_pairs, sort_blk, jnp.int32(0))

    def fire_strip(st):
        stg = jnp.minimum(s0 + st, NSTRIP - 1)
        buf = lax.rem(st, NBUF)
        pltpu.async_copy(
            table_hbm.at[:, pl.ds(stg * 128, 128)],
            strips_v.at[buf],
            sem_strip,
        )

    for j in range(2):
        @pl.when(cnt_s[j] > 0)
        def _():
            fire_strip(jnp.int32(j))

    def drain_out(k):
        def w(i, cc):
            pltpu.make_async_copy(
                rowbuf_v.at[0, pl.ds(0, 1)],
                out_hbm.at[pl.ds(0, 1)],
                sem_out,
            ).wait()
            return cc
        lax.fori_loop(0, k, w, jnp.int32(0))

    def do_strip(st, carry):
        buf = lax.rem(st, NBUF)
        n = cnt_s[st]
        base = off_s[st]

        @pl.when(n > 0)
        def _():
            pltpu.make_async_copy(
                table_hbm.at[:, pl.ds(0, 128)], strips_v.at[0], sem_strip
            ).wait()

        nxt = jnp.minimum(st + 2, SPW)
        @pl.when((st + 2 < SPW) & (cnt_s[nxt] > 0))
        def _():
            fire_strip(st + 2)

        def do_blk(kb, carry2):
            gb, pending = carry2
            drain_out(pending)
            bank = gb & 1
            idblk = sid_v[pl.ds(base + kb * 16, 16)]
            bblk = sb_v[pl.ds(base + kb * 16, 16)]
            for l in range(16):
                @pl.when(kb * 16 + l < n)
                def _():
                    col = idblk[l] & 127
                    colv = jnp.broadcast_to(col, (16,)).astype(jnp.int32)
                    v = [
                        plsc.load_gather(
                            strips_v.at[buf], [lanes + 16 * i, colv])
                        for i in range(NV)
                    ]
                    tot = v[0] * v[0]
                    for i in range(1, NV):
                        tot = tot + v[i] * v[i]
                    for p in perms:
                        tot = tot + tot.at[p].get(mode="promise_in_bounds")
                    ss = tot[0]
                    bits = lax.bitcast_convert_type(ss, jnp.int32)
                    y = lax.bitcast_convert_type(
                        jnp.int32(0x5F3759DF) - (bits >> 1), jnp.float32
                    )
                    for _ in range(3):
                        y = y * (jnp.float32(1.5)
                                 - jnp.float32(0.5) * ss * y * y)
                    inv = jnp.where(
                        ss > jnp.float32(1e-24), y, jnp.float32(1e12))
                    for i in range(NV):
                        rowbuf_v[bank, l, pl.ds(16 * i, 16)] = v[i] * inv
                    pltpu.async_copy(
                        rowbuf_v.at[bank, pl.ds(l, 1)],
                        out_hbm.at[pl.ds(bblk[l], 1)],
                        sem_out,
                    )
            nthis = jnp.minimum(n - kb * 16, jnp.int32(16))
            return (gb + 1, nthis)

        return lax.fori_loop(0, (n + 15) >> 4, do_blk, carry)

    gb_pend = lax.fori_loop(
        0, SPW, do_strip, (jnp.int32(0), jnp.int32(0)))
    drain_out(gb_pend[1])


@jax.jit
def _tower(ids, emb_weight):
    ids32 = ids.astype(jnp.int32)
    table_t = emb_weight.T
    mesh = plsc.VectorSubcoreMesh(core_axis_name="c", subcore_axis_name="s")
    return pl.kernel(
        _tower_body,
        mesh=mesh,
        compiler_params=pltpu.CompilerParams(needs_layout_passes=False),
        out_type=jax.ShapeDtypeStruct((B, D), jnp.float32),
        scratch_types=[
            pltpu.VMEM((B,), jnp.int32),
            pltpu.VMEM((LISTCAP,), jnp.int32),
            pltpu.VMEM((LISTCAP,), jnp.int32),
            pltpu.VMEM((SORTCAP,), jnp.int32),
            pltpu.VMEM((SORTCAP,), jnp.int32),
            pltpu.VMEM((NBUF, D, 128), jnp.float32),
            pltpu.VMEM((2, 16, D), jnp.float32),
            pltpu.VMEM((FILLCAP,), jnp.int32),
            pltpu.SMEM((SPW + 1,), jnp.int32),
            pltpu.SMEM((SPW + 1,), jnp.int32),
            pltpu.SemaphoreType.DMA,
            pltpu.SemaphoreType.DMA,
        ],
    )(ids32, table_t)


def kernel(ids, emb_weight):
    return _tower(ids, emb_weight)


# speedup vs baseline: 1.4279x; 1.4279x over previous
import functools

import jax
import jax.numpy as jnp
from jax import lax
from jax.experimental import pallas as pl
from jax.experimental.pallas import tpu as pltpu
from jax.experimental.pallas import tpu_sc as plsc

B = 16384
D = 64
V = 1000000
NUM_CORES = 2
NUM_SUBCORES = 16
NW = NUM_CORES * NUM_SUBCORES
NV = D // 16
NSTRIP = (V + 127) // 128
SPW = (NSTRIP + NW - 1) // NW
LISTCAP = B + 16
SORTCAP = B + 8 * (SPW + 1) + 16
NBUF = 8
NBLK = B // 16
FILLCAP = 256


def _tower_body(ids_hbm, table_hbm, out_hbm,
                lid_v, sid_v, strips_v, rowbuf_v,
                fill_v, cnt_s, off_s,
                sem_strip, sem_out):
    c = lax.axis_index("c")
    s = lax.axis_index("s")
    wid = s * NUM_CORES + c
    s0 = wid * SPW

    pltpu.sync_copy(ids_hbm, lid_v.at[pl.ds(0, B)])

    lanes = lax.iota(jnp.int32, 16)
    perms = [lanes ^ (1 << k) for k in range(4)]

    zeros16 = jnp.broadcast_to(jnp.int32(0), (16,))
    ones16 = jnp.broadcast_to(jnp.int32(1), (16,))
    for i in range(FILLCAP // 16):
        fill_v[pl.ds(16 * i, 16)] = zeros16

    def scan_blk(g, k):
        idvec = lid_v[pl.ds(g * 16, 16)]
        stripv = idvec >> 7
        m = (stripv >= s0) & (stripv < s0 + SPW)
        stl = jnp.where(m, stripv - s0, jnp.int32(SPW))
        plsc.addupdate_scatter(fill_v, [stl], ones16, mask=m)
        pack = ((idvec - s0 * 128) << 14) | (lanes + g * 16)
        skeys, _, om = plsc.sort_key_val(pack, pack, mask=m)
        cnt = plsc.all_reduce_population_count(m)
        c = cnt if getattr(cnt, "ndim", 0) == 0 else cnt[0]
        plsc.store_scatter(lid_v, [k + lanes], skeys, mask=om)
        return k + c
    num_pairs = lax.fori_loop(0, NBLK, scan_blk, jnp.int32(0), unroll=8)

    nblk_pairs = (num_pairs + 15) >> 4

    def offs_blk(kb, running):
        cvec = fill_v[pl.ds(kb * 16, 16)]
        for l in range(16):
            st = kb * 16 + l
            @pl.when(st < SPW)
            def _():
                cnt_s[st] = cvec[l]
                off_s[st] = running
            plsc.store_scatter(
                fill_v, [jnp.broadcast_to(st, (16,)).astype(jnp.int32)],
                jnp.broadcast_to(running, (16,)), mask=lanes == 0)
            running = jnp.where(
                st < SPW,
                running + ((cvec[l] + 7) & ~jnp.int32(7)),
                running,
            )
        return running
    lax.fori_loop(0, (SPW + 15) // 16, offs_blk, jnp.int32(0))

    def sort_blk(kb, carry):
        packblk = lid_v[pl.ds(kb * 16, 16)]
        valid = (kb * 16 + lanes) < num_pairs
        stl = jnp.where(valid, packblk >> 21, jnp.int32(SPW))
        fill = plsc.load_gather(fill_v, [stl])
        rank = zeros16
        for k2 in range(1, 16):
            prev = stl.at[jnp.maximum(lanes - k2, 0)].get(
                mode="promise_in_bounds")
            eq = (prev == stl) & (lanes >= k2)
            rank = rank + jnp.where(eq, jnp.int32(1), jnp.int32(0))
        pos = fill + rank
        plsc.store_scatter(sid_v, [pos], packblk, mask=valid)
        plsc.addupdate_scatter(fill_v, [stl], ones16, mask=valid)
        return carry
    lax.fori_loop(0, nblk_pairs, sort_blk, jnp.int32(0))

    def fire_strip(st):
        stg = jnp.minimum(s0 + st, NSTRIP - 1)
        buf = lax.rem(st, NBUF)
        pltpu.async_copy(
            table_hbm.at[:, pl.ds(stg * 128, 128)],
            strips_v.at[buf],
            sem_strip,
        )

    for j in range(NBUF - 1):
        @pl.when(cnt_s[j] > 0)
        def _():
            fire_strip(jnp.int32(j))

    def drain_out(k):
        def w(i, cc):
            pltpu.make_async_copy(
                rowbuf_v.at[0, pl.ds(0, 1)],
                out_hbm.at[pl.ds(0, 1)],
                sem_out,
            ).wait()
            return cc
        lax.fori_loop(0, k, w, jnp.int32(0))

    def do_strip(st, carry):
        buf = lax.rem(st, NBUF)
        n = cnt_s[st]
        base = off_s[st]

        @pl.when(n > 0)
        def _():
            pltpu.make_async_copy(
                table_hbm.at[:, pl.ds(0, 128)], strips_v.at[0], sem_strip
            ).wait()

        nxt = jnp.minimum(st + NBUF - 1, SPW)
        @pl.when((st + NBUF - 1 < SPW) & (cnt_s[nxt] > 0))
        def _():
            fire_strip(st + NBUF - 1)

        def do_blk(kb, carry2):
            gb, pending = carry2
            drain_out(pending)
            bank = gb & 1
            packblk = sid_v[pl.ds(base + kb * 16, 16)]
            for l in range(16):
                @pl.when(kb * 16 + l < n)
                def _():
                    col = (packblk[l] >> 14) & 127
                    colv = jnp.broadcast_to(col, (16,)).astype(jnp.int32)
                    v = [
                        plsc.load_gather(
                            strips_v.at[buf], [lanes + 16 * i, colv])
                        for i in range(NV)
                    ]
                    tot = v[0] * v[0]
                    for i in range(1, NV):
                        tot = tot + v[i] * v[i]
                    for p in perms:
                        tot = tot + tot.at[p].get(mode="promise_in_bounds")
                    ss = tot[0]
                    bits = lax.bitcast_convert_type(ss, jnp.int32)
                    y = lax.bitcast_convert_type(
                        jnp.int32(0x5F3759DF) - (bits >> 1), jnp.float32
                    )
                    for _ in range(3):
                        y = y * (jnp.float32(1.5)
                                 - jnp.float32(0.5) * ss * y * y)
                    inv = jnp.where(
                        ss > jnp.float32(1e-24), y, jnp.float32(1e12))
                    for i in range(NV):
                        rowbuf_v[bank, l, pl.ds(16 * i, 16)] = v[i] * inv
                    pltpu.async_copy(
                        rowbuf_v.at[bank, pl.ds(l, 1)],
                        out_hbm.at[pl.ds(packblk[l] & 16383, 1)],
                        sem_out,
                    )
            nthis = jnp.minimum(n - kb * 16, jnp.int32(16))
            return (gb + 1, nthis)

        return lax.fori_loop(0, (n + 15) >> 4, do_blk, carry)

    gb_pend = lax.fori_loop(
        0, SPW, do_strip, (jnp.int32(0), jnp.int32(0)))
    drain_out(gb_pend[1])


@jax.jit
def _tower(ids, emb_weight):
    ids32 = ids.astype(jnp.int32)
    table_t = emb_weight.T
    mesh = plsc.VectorSubcoreMesh(core_axis_name="c", subcore_axis_name="s")
    return pl.kernel(
        _tower_body,
        mesh=mesh,
        compiler_params=pltpu.CompilerParams(needs_layout_passes=False),
        out_type=jax.ShapeDtypeStruct((B, D), jnp.float32),
        scratch_types=[
            pltpu.VMEM((LISTCAP,), jnp.int32),
            pltpu.VMEM((SORTCAP,), jnp.int32),
            pltpu.VMEM((NBUF, D, 128), jnp.float32),
            pltpu.VMEM((2, 16, D), jnp.float32),
            pltpu.VMEM((FILLCAP,), jnp.int32),
            pltpu.SMEM((SPW + 1,), jnp.int32),
            pltpu.SMEM((SPW + 1,), jnp.int32),
            pltpu.SemaphoreType.DMA,
            pltpu.SemaphoreType.DMA,
        ],
    )(ids32, table_t)


def kernel(ids, emb_weight):
    return _tower(ids, emb_weight)
